# baseline (speedup 1.0000x reference)
import functools

import jax
import jax.numpy as jnp
from jax import lax
from jax.experimental import pallas as pl
from jax.experimental.pallas import tpu as pltpu
from jax.experimental.pallas import tpu_sc as plsc

_E = 64
_ROWS = 16384
_IDX_N = 32768
_NC = 2
_NS = 16
_NW = _NC * _NS
_IPW = _IDX_N // _NW
_L = 16

_GRID = 16
_BLK = _ROWS // _GRID


def _sc_hist_body(idx_hbm, out_hbm, idx_v, hist_v, out_v):
    wid = lax.axis_index("s") * _NC + lax.axis_index("c")
    pltpu.sync_copy(idx_hbm.at[pl.ds(wid * _IPW, _IPW)], idx_v)

    zero16 = jnp.zeros((_L,), jnp.float32)
    for i in range(_L * _E // _L):
        hist_v[pl.ds(i * _L, _L)] = zero16

    lane_base = lax.broadcasted_iota(jnp.int32, (_L,), 0) * _E
    ones = jnp.ones((_L,), jnp.float32)
    for i in range(_IPW // _L):
        v = idx_v[pl.ds(i * _L, _L)]
        plsc.addupdate_scatter(hist_v, [lane_base + v], ones)

    for j in range(_E // _L):
        acc = zero16
        for r in range(_L):
            acc = acc + hist_v[pl.ds(r * _E + j * _L, _L)]
        out_v[pl.ds(j * _L, _L)] = acc

    pltpu.sync_copy(out_v, out_hbm.at[pl.ds(wid * _E, _E)])


@functools.cache
def _sc_hist():
    return functools.partial(
        pl.kernel,
        mesh=plsc.VectorSubcoreMesh(core_axis_name="c", subcore_axis_name="s"),
        out_type=jax.ShapeDtypeStruct((_NW * _E,), jnp.float32),
        scratch_types=[
            pltpu.VMEM((_IPW,), jnp.int32),
            pltpu.VMEM((_L * _E,), jnp.float32),
            pltpu.VMEM((_E,), jnp.float32),
        ],
        compiler_params=pltpu.CompilerParams(needs_layout_passes=False),
    )(_sc_hist_body)


def _tc_body(logits_ref, hist_ref, loss_ref, acc_ref):
    step = pl.program_id(0)

    @pl.when(step == 0)
    def _init():
        acc_ref[...] = jnp.zeros_like(acc_ref)

    x = logits_ref[...]
    m = jnp.max(x, axis=1, keepdims=True)
    e = jnp.exp(x - m)
    s = jnp.sum(e, axis=1, keepdims=True)
    acc_ref[...] += jnp.sum(e / s, axis=0, keepdims=True)

    @pl.when(step == _GRID - 1)
    def _fin():
        counts = jnp.sum(hist_ref[...], axis=0, keepdims=True)
        total = jnp.sum(acc_ref[...] * counts)
        loss_ref[0, 0] = total * (_E / (_ROWS * _ROWS))


def _tc_call(logits, hist):
    return pl.pallas_call(
        _tc_body,
        grid=(_GRID,),
        in_specs=[
            pl.BlockSpec((_BLK, _E), lambda i: (i, 0)),
            pl.BlockSpec((_NW, _E), lambda i: (0, 0)),
        ],
        out_specs=pl.BlockSpec(memory_space=pltpu.SMEM),
        out_shape=jax.ShapeDtypeStruct((1, 1), jnp.float32),
        scratch_shapes=[pltpu.VMEM((1, _E), jnp.float32)],
    )(logits, hist)


def kernel(router_logits, expert_indices):
    logits = router_logits.reshape(_ROWS, _E)
    idx = expert_indices.astype(jnp.int32).reshape(_IDX_N)
    hist = _sc_hist()(idx).reshape(_NW, _E)
    return _tc_call(logits, hist)[0, 0]
